# baseline (speedup 1.0000x reference)
import jax
import jax.numpy as jnp
from jax import lax
from jax.experimental import pallas as pl
from jax.experimental.pallas import tpu as pltpu
from jax.experimental.pallas import tpu_sc as plsc

B, L, NBINS, D = 4, 512, 65, 128
NC, NS, LANES = 2, 16, 16
NW = NC * NS
ROWS = B * L * L
ROWS_PER_W = ROWS // NW
CHUNK = 128
CHUNKS_PER_W = ROWS_PER_W // CHUNK


def _relpos_body(table_hbm, res_hbm, rep_hbm, out_hbm, res_v, rep_v, idx_v, rows_v, gsem, osem):
    wid = lax.axis_index("s") * NC + lax.axis_index("c")
    pltpu.sync_copy(res_hbm, res_v)
    pltpu.sync_copy(rep_hbm.at[pl.ds(wid * (ROWS_PER_W // L) * LANES,
                                     (ROWS_PER_W // L) * LANES)], rep_v)

    for k in range(CHUNK // LANES):
        idx_v[pl.ds(k * LANES, LANES)] = jnp.zeros((LANES,), jnp.int32)

    def chunk_body(c, _):
        r0 = wid * ROWS_PER_W + c * CHUNK
        bi = lax.shift_right_logical(r0, 9)
        jbase = lax.shift_left(lax.shift_right_logical(r0, 18), 9) + jnp.bitwise_and(
            r0, L - 1
        )
        tloc = bi - wid * (ROWS_PER_W // L)
        ri = rep_v[pl.ds(tloc * LANES, LANES)]
        if True:
            pass
        else:
            for k in range(CHUNK // LANES):
                rj = res_v[pl.ds(jbase + k * LANES, LANES)]
                d = jnp.clip(ri - rj, -32, 32) + 32
                idx_v[pl.ds(k * LANES, LANES)] = d
        pltpu.async_copy(table_hbm.at[idx_v], rows_v, gsem).wait()
        pltpu.async_copy(rows_v, out_hbm.at[pl.ds(r0, CHUNK)], osem).wait()
        return ()

    lax.fori_loop(0, CHUNKS_PER_W, chunk_body, ())


@jax.jit
def kernel(res_id, W, b):
    table = W.T + b[None, :]
    res_flat = res_id.reshape(-1).astype(jnp.int32)
    res_rep = jnp.broadcast_to(res_flat[:, None], (B * L, LANES)).reshape(-1)
    mesh = plsc.VectorSubcoreMesh(
        core_axis_name="c", subcore_axis_name="s", num_cores=NC, num_subcores=NS
    )
    out = pl.kernel(
        _relpos_body,
        out_type=jax.ShapeDtypeStruct((ROWS, D), jnp.float32),
        mesh=mesh,
        scratch_types=[
            pltpu.VMEM((B * L,), jnp.int32),
            pltpu.VMEM(((ROWS_PER_W // L) * LANES,), jnp.int32),
            pltpu.VMEM((CHUNK,), jnp.int32),
            pltpu.VMEM((CHUNK, D), jnp.float32),
            pltpu.SemaphoreType.DMA,
            pltpu.SemaphoreType.DMA,
        ],
    )(table, res_flat, res_rep)
    return out.reshape(B, L, L, D)


# speedup vs baseline: 40.9411x; 40.9411x over previous
import jax
import jax.numpy as jnp
from jax import lax
from jax.experimental import pallas as pl
from jax.experimental.pallas import tpu as pltpu
from jax.experimental.pallas import tpu_sc as plsc

B, L, NBINS, D = 4, 512, 65, 128
NC, NS, LANES = 2, 16, 16
NW = NC * NS
ROWS = B * L * L
ROWS_PER_W = ROWS // NW
CHUNK = 128
CHUNKS_PER_W = ROWS_PER_W // CHUNK


def _relpos_body(table_hbm, res_hbm, rep_hbm, out_hbm,
                 table_v, res_v, rep_v, idx_v, rows0, rows1, osem0, osem1):
    wid = lax.axis_index("s") * NC + lax.axis_index("c")
    pltpu.sync_copy(table_hbm, table_v)
    pltpu.sync_copy(res_hbm, res_v)
    pltpu.sync_copy(rep_hbm.at[pl.ds(wid * (ROWS_PER_W // L) * LANES,
                                     (ROWS_PER_W // L) * LANES)], rep_v)
    rows = (rows0, rows1)
    osem = (osem0, osem1)

    def chunk_pair_body(c2, _):
        for p in range(2):
            c = c2 * 2 + p
            r0 = wid * ROWS_PER_W + c * CHUNK
            bi = lax.shift_right_logical(r0, 9)
            jbase = lax.shift_left(lax.shift_right_logical(r0, 18), 9) \
                + jnp.bitwise_and(r0, L - 1)
            tloc = bi - wid * (ROWS_PER_W // L)
            ri = rep_v[pl.ds(tloc * LANES, LANES)]
            for k in range(CHUNK // LANES):
                rj = res_v[pl.ds(jbase + k * LANES, LANES)]
                d = jnp.clip(ri - rj, -32, 32) + 32
                idx_v[pl.ds(k * LANES, LANES)] = lax.shift_left(d, 7)

            @pl.when(c2 > 0)
            def _wait_prev():
                pltpu.make_async_copy(
                    rows[p], out_hbm.at[pl.ds(0, CHUNK * D)], osem[p]).wait()

            def group_body(g, _):
                blk = idx_v[pl.ds(g * LANES, LANES)]
                for t in range(LANES):
                    src = blk[t]
                    dst = lax.shift_left(g * LANES + t, 7)
                    for k in range(D // LANES):
                        rows[p][pl.ds(dst + k * LANES, LANES)] = \
                            table_v[pl.ds(src + k * LANES, LANES)]
                return ()

            lax.fori_loop(0, CHUNK // LANES, group_body, ())
            pltpu.async_copy(
                rows[p], out_hbm.at[pl.ds(r0 * D, CHUNK * D)], osem[p])
        return ()

    lax.fori_loop(0, CHUNKS_PER_W // 2, chunk_pair_body, ())
    for p in range(2):
        pltpu.make_async_copy(
            rows[p], out_hbm.at[pl.ds(0, CHUNK * D)], osem[p]).wait()


@jax.jit
def kernel(res_id, W, b):
    table = (W.T + b[None, :]).reshape(-1)
    res_flat = res_id.reshape(-1).astype(jnp.int32)
    res_rep = jnp.broadcast_to(res_flat[:, None], (B * L, LANES)).reshape(-1)
    mesh = plsc.VectorSubcoreMesh(
        core_axis_name="c", subcore_axis_name="s", num_cores=NC, num_subcores=NS
    )
    out = pl.kernel(
        _relpos_body,
        out_type=jax.ShapeDtypeStruct((ROWS * D,), jnp.float32),
        mesh=mesh,
        scratch_types=[
            pltpu.VMEM((NBINS * D,), jnp.float32),
            pltpu.VMEM((B * L,), jnp.int32),
            pltpu.VMEM(((ROWS_PER_W // L) * LANES,), jnp.int32),
            pltpu.VMEM((CHUNK,), jnp.int32),
            pltpu.VMEM((CHUNK * D,), jnp.float32),
            pltpu.VMEM((CHUNK * D,), jnp.float32),
            pltpu.SemaphoreType.DMA,
            pltpu.SemaphoreType.DMA,
        ],
    )(table, res_flat, res_rep)
    return out.reshape(B, L, L, D)


# speedup vs baseline: 91.0362x; 2.2236x over previous
import jax
import jax.numpy as jnp
from jax import lax
from jax.experimental import pallas as pl
from jax.experimental.pallas import tpu as pltpu
from jax.experimental.pallas import tpu_sc as plsc

B, L, NBINS, D = 4, 512, 65, 128
NC, NS, LANES = 2, 16, 16
NW = NC * NS
ROWS = B * L * L
ROWS_PER_W = ROWS // NW
CHUNK = 256
CHUNKS_PER_W = ROWS_PER_W // CHUNK


def _relpos_body(table_hbm, res_hbm, rep_hbm, out_hbm,
                 table_v, res_v, rep_v, idx_v, rows0, rows1, osem0, osem1):
    wid = lax.axis_index("s") * NC + lax.axis_index("c")
    pltpu.sync_copy(table_hbm, table_v)
    pltpu.sync_copy(res_hbm, res_v)
    pltpu.sync_copy(rep_hbm.at[pl.ds(wid * (ROWS_PER_W // L) * LANES,
                                     (ROWS_PER_W // L) * LANES)], rep_v)
    rows = (rows0, rows1)
    osem = (osem0, osem1)

    def chunk_pair_body(c2, _):
        for p in range(2):
            c = c2 * 2 + p
            r0 = wid * ROWS_PER_W + c * CHUNK
            bi = lax.shift_right_logical(r0, 9)
            jbase = lax.shift_left(lax.shift_right_logical(r0, 18), 9) \
                + jnp.bitwise_and(r0, L - 1)
            tloc = bi - wid * (ROWS_PER_W // L)
            ri = rep_v[pl.ds(tloc * LANES, LANES)]
            for k in range(CHUNK // LANES):
                rj = res_v[pl.ds(jbase + k * LANES, LANES)]
                d = jnp.clip(ri - rj, -32, 32) + 32
                idx_v[pl.ds(k * LANES, LANES)] = lax.shift_left(d, 7)

            @pl.when(c2 > 0)
            def _wait_prev():
                pltpu.make_async_copy(
                    rows[p], out_hbm.at[pl.ds(0, CHUNK * D)], osem[p]).wait()

            @plsc.parallel_loop(0, CHUNK // LANES, unroll=2)
            def group_body(g):
                blk = idx_v[pl.ds(g * LANES, LANES)]
                for t in range(LANES):
                    src = blk[t]
                    dst = lax.shift_left(g * LANES + t, 7)
                    for k in range(D // LANES):
                        rows[p][pl.ds(dst + k * LANES, LANES)] = \
                            table_v[pl.ds(src + k * LANES, LANES)]
            pltpu.async_copy(
                rows[p], out_hbm.at[pl.ds(r0 * D, CHUNK * D)], osem[p])
        return ()

    lax.fori_loop(0, CHUNKS_PER_W // 2, chunk_pair_body, ())
    for p in range(2):
        pltpu.make_async_copy(
            rows[p], out_hbm.at[pl.ds(0, CHUNK * D)], osem[p]).wait()


@jax.jit
def kernel(res_id, W, b):
    table = (W.T + b[None, :]).reshape(-1)
    res_flat = res_id.reshape(-1).astype(jnp.int32)
    res_rep = jnp.broadcast_to(res_flat[:, None], (B * L, LANES)).reshape(-1)
    mesh = plsc.VectorSubcoreMesh(
        core_axis_name="c", subcore_axis_name="s", num_cores=NC, num_subcores=NS
    )
    out = pl.kernel(
        _relpos_body,
        out_type=jax.ShapeDtypeStruct((ROWS * D,), jnp.float32),
        mesh=mesh,
        scratch_types=[
            pltpu.VMEM((NBINS * D,), jnp.float32),
            pltpu.VMEM((B * L,), jnp.int32),
            pltpu.VMEM(((ROWS_PER_W // L) * LANES,), jnp.int32),
            pltpu.VMEM((CHUNK,), jnp.int32),
            pltpu.VMEM((CHUNK * D,), jnp.float32),
            pltpu.VMEM((CHUNK * D,), jnp.float32),
            pltpu.SemaphoreType.DMA,
            pltpu.SemaphoreType.DMA,
        ],
    )(table, res_flat, res_rep)
    return out.reshape(B, L, L, D)
